# baseline (speedup 1.0000x reference)
import functools

import jax
import jax.numpy as jnp
from jax import lax
from jax.experimental import pallas as pl
from jax.experimental.pallas import tpu as pltpu
from jax.experimental.pallas import tpu_sc as plsc

B = 4096
N_NUM = 13
N_CAT = 26
VOCAB = 1000
EMB = 50
R = B * N_CAT
EMBP = 64
TB = 512
EPS = 1e-5


def _build_gather():
    info = plsc.get_sparse_core_info()
    nc, ns = info.num_cores, info.num_subcores
    nw = nc * ns
    per_w = R // nw
    ch = 128
    nch = per_w // ch
    assert per_w * nw == R and nch * ch == per_w

    mesh = plsc.VectorSubcoreMesh(core_axis_name="c", subcore_axis_name="s")

    @functools.partial(
        pl.kernel,
        out_type=jax.ShapeDtypeStruct((R, EMBP), jnp.float32),
        mesh=mesh,
        compiler_params=pltpu.CompilerParams(use_tc_tiling_on_sc=False),
        scratch_types=[
            pltpu.VMEM((per_w,), jnp.int32),
            pltpu.VMEM((ch, EMBP), jnp.float32),
            pltpu.VMEM((ch, EMBP), jnp.float32),
            pltpu.SemaphoreType.DMA,
            pltpu.SemaphoreType.DMA,
        ],
    )
    def gather(table_hbm, idx_hbm, out_hbm, idx_v, rows_a, rows_b, sem_a, sem_b):
        wid = lax.axis_index("s") * nc + lax.axis_index("c")
        base = wid * per_w
        pltpu.sync_copy(idx_hbm.at[pl.ds(base, per_w)], idx_v)

        def body(i, carry):
            c0 = i * 2
            d_a = pltpu.async_copy(
                table_hbm.at[idx_v.at[pl.ds(c0 * ch, ch)]], rows_a, sem_a)
            d_b = pltpu.async_copy(
                table_hbm.at[idx_v.at[pl.ds((c0 + 1) * ch, ch)]], rows_b, sem_b)
            d_a.wait()
            pltpu.sync_copy(rows_a, out_hbm.at[pl.ds(base + c0 * ch, ch)])
            d_b.wait()
            pltpu.sync_copy(rows_b, out_hbm.at[pl.ds(base + (c0 + 1) * ch, ch)])
            return carry

        lax.fori_loop(0, nch // 2, body, 0)

    return gather


_gather_cache = []


def _gather(table, gidx):
    if not _gather_cache:
        _gather_cache.append(_build_gather())
    return _gather_cache[0](table, gidx)


def _layer1_body(xnum_ref, emb_ref, w1a_ref, w1b_ref, b1_ref, h_ref, s_ref, q_ref):
    i = pl.program_id(0)
    e = jnp.concatenate([emb_ref[f] for f in range(N_CAT)], axis=-1)
    h = (jnp.dot(xnum_ref[...].astype(jnp.bfloat16),
                 w1a_ref[...].astype(jnp.bfloat16),
                 preferred_element_type=jnp.float32)
         + jnp.dot(e.astype(jnp.bfloat16), w1b_ref[...].astype(jnp.bfloat16),
                   preferred_element_type=jnp.float32)
         + b1_ref[...])
    h_ref[...] = h

    @pl.when(i == 0)
    def _():
        s_ref[...] = jnp.zeros_like(s_ref)
        q_ref[...] = jnp.zeros_like(q_ref)

    s_ref[...] += jnp.sum(h, axis=0, keepdims=True)
    q_ref[...] += jnp.sum(h * h, axis=0, keepdims=True)


def _mid_body(h_ref, s_in, q_in, g_ref, bt_ref, w_ref, b_ref, o_ref, s_ref, q_ref):
    i = pl.program_id(0)
    mu = s_in[...] * (1.0 / B)
    var = q_in[...] * (1.0 / B) - mu * mu
    a = g_ref[...] * lax.rsqrt(var + EPS)
    c = bt_ref[...] - mu * a
    x = jnp.maximum(h_ref[...] * a + c, 0.0)
    h = jnp.dot(x.astype(jnp.bfloat16), w_ref[...].astype(jnp.bfloat16),
                preferred_element_type=jnp.float32) + b_ref[...]
    o_ref[...] = h

    @pl.when(i == 0)
    def _():
        s_ref[...] = jnp.zeros_like(s_ref)
        q_ref[...] = jnp.zeros_like(q_ref)

    s_ref[...] += jnp.sum(h, axis=0, keepdims=True)
    q_ref[...] += jnp.sum(h * h, axis=0, keepdims=True)


def _final_body(h_ref, s_in, q_in, g_ref, bt_ref, wo_ref, bo_ref, o_ref):
    mu = s_in[...] * (1.0 / B)
    var = q_in[...] * (1.0 / B) - mu * mu
    a = g_ref[...] * lax.rsqrt(var + EPS)
    c = bt_ref[...] - mu * a
    x = jnp.maximum(h_ref[...] * a + c, 0.0)
    o_ref[...] = jnp.sum(x * wo_ref[...], axis=1, keepdims=True) + bo_ref[0, 0]


def _full(shape):
    return pl.BlockSpec(shape, lambda i: (0, 0))


def _layer1(xnum, emb, w1a, w1b, b1):
    d = w1a.shape[1]
    grid = B // TB
    return pl.pallas_call(
        _layer1_body,
        grid=(grid,),
        in_specs=[
            pl.BlockSpec((TB, N_NUM), lambda i: (i, 0)),
            pl.BlockSpec((N_CAT, TB, EMBP), lambda i: (0, i, 0)),
            _full(w1a.shape),
            _full(w1b.shape),
            _full((1, d)),
        ],
        out_specs=[
            pl.BlockSpec((TB, d), lambda i: (i, 0)),
            _full((1, d)),
            _full((1, d)),
        ],
        out_shape=[
            jax.ShapeDtypeStruct((B, d), jnp.float32),
            jax.ShapeDtypeStruct((1, d), jnp.float32),
            jax.ShapeDtypeStruct((1, d), jnp.float32),
        ],
    )(xnum, emb, w1a, w1b, b1)


def _mid(h, s, q, g, bt, w, b):
    d_in, d_out = w.shape
    grid = B // TB
    return pl.pallas_call(
        _mid_body,
        grid=(grid,),
        in_specs=[
            pl.BlockSpec((TB, d_in), lambda i: (i, 0)),
            _full((1, d_in)),
            _full((1, d_in)),
            _full((1, d_in)),
            _full((1, d_in)),
            _full((d_in, d_out)),
            _full((1, d_out)),
        ],
        out_specs=[
            pl.BlockSpec((TB, d_out), lambda i: (i, 0)),
            _full((1, d_out)),
            _full((1, d_out)),
        ],
        out_shape=[
            jax.ShapeDtypeStruct((B, d_out), jnp.float32),
            jax.ShapeDtypeStruct((1, d_out), jnp.float32),
            jax.ShapeDtypeStruct((1, d_out), jnp.float32),
        ],
    )(h, s, q, g, bt, w, b)


def _final(h, s, q, g, bt, wo, bo):
    d_in = h.shape[1]
    grid = B // TB
    return pl.pallas_call(
        _final_body,
        grid=(grid,),
        in_specs=[
            pl.BlockSpec((TB, d_in), lambda i: (i, 0)),
            _full((1, d_in)),
            _full((1, d_in)),
            _full((1, d_in)),
            _full((1, d_in)),
            _full((1, d_in)),
            _full((1, 1)),
        ],
        out_specs=pl.BlockSpec((TB, 1), lambda i: (i, 0)),
        out_shape=jax.ShapeDtypeStruct((B, 1), jnp.float32),
    )(h, s, q, g, bt, wo, bo)


def kernel(xb, emb_tables, W1, b1, g1, bt1, W2, b2, g2, bt2, W3, b3, g3, bt3, Wo, bo):
    cats = xb[:, N_NUM:N_NUM + N_CAT].astype(jnp.int32)
    gidx = (cats.T + (jnp.arange(N_CAT, dtype=jnp.int32) * VOCAB)[:, None]).reshape(-1)
    table = jnp.pad(emb_tables.reshape(N_CAT * VOCAB, EMB),
                    ((0, 0), (0, EMBP - EMB)))

    emb = _gather(table, gidx).reshape(N_CAT, B, EMBP)
    xnum = xb[:, :N_NUM]

    w1b = jnp.pad(W1[N_NUM:].reshape(N_CAT, EMB, -1),
                  ((0, 0), (0, EMBP - EMB), (0, 0))).reshape(N_CAT * EMBP, -1)
    h1, s1, q1 = _layer1(xnum, emb, W1[:N_NUM], w1b, b1.reshape(1, -1))
    h2, s2, q2 = _mid(h1, s1, q1, g1.reshape(1, -1), bt1.reshape(1, -1), W2,
                      b2.reshape(1, -1))
    h3, s3, q3 = _mid(h2, s2, q2, g2.reshape(1, -1), bt2.reshape(1, -1), W3,
                      b3.reshape(1, -1))
    out = _final(h3, s3, q3, g3.reshape(1, -1), bt3.reshape(1, -1),
                 Wo.reshape(1, -1), bo.reshape(1, 1))
    return out.reshape(B)


# speedup vs baseline: 1.1030x; 1.1030x over previous
import functools

import jax
import jax.numpy as jnp
from jax import lax
from jax.experimental import pallas as pl
from jax.experimental.pallas import tpu as pltpu
from jax.experimental.pallas import tpu_sc as plsc

B = 4096
N_NUM = 13
N_CAT = 26
VOCAB = 1000
EMB = 50
R = B * N_CAT
EMBP = 64
TB = 512
EPS = 1e-5


def _build_gather():
    info = plsc.get_sparse_core_info()
    nc, ns = info.num_cores, info.num_subcores
    nw = nc * ns
    per_w = R // nw
    ch = 128
    nch = per_w // ch
    assert per_w * nw == R and nch * ch == per_w

    mesh = plsc.VectorSubcoreMesh(core_axis_name="c", subcore_axis_name="s")

    @functools.partial(
        pl.kernel,
        out_type=jax.ShapeDtypeStruct((R, EMBP), jnp.float32),
        mesh=mesh,
        compiler_params=pltpu.CompilerParams(use_tc_tiling_on_sc=False),
        scratch_types=[
            pltpu.VMEM((per_w,), jnp.int32),
            pltpu.VMEM((ch, EMBP), jnp.float32),
            pltpu.VMEM((ch, EMBP), jnp.float32),
            pltpu.SemaphoreType.DMA,
            pltpu.SemaphoreType.DMA,
        ],
    )
    def gather(table_hbm, idx_hbm, out_hbm, idx_v, rows_a, rows_b, sem_a, sem_b):
        wid = lax.axis_index("s") * nc + lax.axis_index("c")
        base = wid * per_w
        pltpu.sync_copy(idx_hbm.at[pl.ds(base, per_w)], idx_v)

        def body(i, carry):
            c0 = i * 2
            d_a = pltpu.async_copy(
                table_hbm.at[idx_v.at[pl.ds(c0 * ch, ch)]], rows_a, sem_a)
            d_b = pltpu.async_copy(
                table_hbm.at[idx_v.at[pl.ds((c0 + 1) * ch, ch)]], rows_b, sem_b)
            d_a.wait()
            pltpu.sync_copy(rows_a, out_hbm.at[pl.ds(base + c0 * ch, ch)])
            d_b.wait()
            pltpu.sync_copy(rows_b, out_hbm.at[pl.ds(base + (c0 + 1) * ch, ch)])
            return carry

        lax.fori_loop(0, nch // 2, body, 0)

    return gather


_gather_cache = []


def _gather(table, gidx):
    if not _gather_cache:
        _gather_cache.append(_build_gather())
    return _gather_cache[0](table, gidx)


def _layer1_body(xnum_ref, emb_ref, w1a_ref, w1b_ref, b1_ref, h_ref, s_ref, q_ref,
                 wp_ref):
    i = pl.program_id(0)

    @pl.when(i == 0)
    def _():
        wp_ref[...] = jnp.zeros_like(wp_ref)
        for f in range(N_CAT):
            wp_ref[pl.ds(f * EMBP, EMB), :] = (
                w1b_ref[pl.ds(f * EMB, EMB), :].astype(jnp.bfloat16))

    h = (jnp.dot(xnum_ref[...].astype(jnp.bfloat16),
                 w1a_ref[...].astype(jnp.bfloat16),
                 preferred_element_type=jnp.float32)
         + jnp.dot(emb_ref[...].astype(jnp.bfloat16), wp_ref[...],
                   preferred_element_type=jnp.float32)
         + b1_ref[...])
    h_ref[...] = h

    @pl.when(i == 0)
    def _():
        s_ref[...] = jnp.zeros_like(s_ref)
        q_ref[...] = jnp.zeros_like(q_ref)

    s_ref[...] += jnp.sum(h, axis=0, keepdims=True)
    q_ref[...] += jnp.sum(h * h, axis=0, keepdims=True)


def _mid_body(h_ref, s_in, q_in, g_ref, bt_ref, w_ref, b_ref, o_ref, s_ref, q_ref):
    i = pl.program_id(0)
    mu = s_in[...] * (1.0 / B)
    var = q_in[...] * (1.0 / B) - mu * mu
    a = g_ref[...] * lax.rsqrt(var + EPS)
    c = bt_ref[...] - mu * a
    x = jnp.maximum(h_ref[...] * a + c, 0.0)
    h = jnp.dot(x.astype(jnp.bfloat16), w_ref[...].astype(jnp.bfloat16),
                preferred_element_type=jnp.float32) + b_ref[...]
    o_ref[...] = h

    @pl.when(i == 0)
    def _():
        s_ref[...] = jnp.zeros_like(s_ref)
        q_ref[...] = jnp.zeros_like(q_ref)

    s_ref[...] += jnp.sum(h, axis=0, keepdims=True)
    q_ref[...] += jnp.sum(h * h, axis=0, keepdims=True)


def _final_body(h_ref, s_in, q_in, g_ref, bt_ref, wo_ref, bo_ref, o_ref):
    mu = s_in[...] * (1.0 / B)
    var = q_in[...] * (1.0 / B) - mu * mu
    a = g_ref[...] * lax.rsqrt(var + EPS)
    c = bt_ref[...] - mu * a
    x = jnp.maximum(h_ref[...] * a + c, 0.0)
    o_ref[...] = jnp.sum(x * wo_ref[...], axis=1, keepdims=True) + bo_ref[0, 0]


def _full(shape):
    return pl.BlockSpec(shape, lambda i: (0, 0))


def _layer1(xnum, emb, w1a, w1b, b1):
    d = w1a.shape[1]
    grid = B // TB
    return pl.pallas_call(
        _layer1_body,
        grid=(grid,),
        in_specs=[
            pl.BlockSpec((TB, N_NUM), lambda i: (i, 0)),
            pl.BlockSpec((TB, N_CAT * EMBP), lambda i: (i, 0)),
            _full(w1a.shape),
            _full(w1b.shape),
            _full((1, d)),
        ],
        out_specs=[
            pl.BlockSpec((TB, d), lambda i: (i, 0)),
            _full((1, d)),
            _full((1, d)),
        ],
        out_shape=[
            jax.ShapeDtypeStruct((B, d), jnp.float32),
            jax.ShapeDtypeStruct((1, d), jnp.float32),
            jax.ShapeDtypeStruct((1, d), jnp.float32),
        ],
        scratch_shapes=[pltpu.VMEM((N_CAT * EMBP, d), jnp.bfloat16)],
    )(xnum, emb, w1a, w1b, b1)


def _mid(h, s, q, g, bt, w, b):
    d_in, d_out = w.shape
    grid = B // TB
    return pl.pallas_call(
        _mid_body,
        grid=(grid,),
        in_specs=[
            pl.BlockSpec((TB, d_in), lambda i: (i, 0)),
            _full((1, d_in)),
            _full((1, d_in)),
            _full((1, d_in)),
            _full((1, d_in)),
            _full((d_in, d_out)),
            _full((1, d_out)),
        ],
        out_specs=[
            pl.BlockSpec((TB, d_out), lambda i: (i, 0)),
            _full((1, d_out)),
            _full((1, d_out)),
        ],
        out_shape=[
            jax.ShapeDtypeStruct((B, d_out), jnp.float32),
            jax.ShapeDtypeStruct((1, d_out), jnp.float32),
            jax.ShapeDtypeStruct((1, d_out), jnp.float32),
        ],
    )(h, s, q, g, bt, w, b)


def _final(h, s, q, g, bt, wo, bo):
    d_in = h.shape[1]
    grid = B // TB
    return pl.pallas_call(
        _final_body,
        grid=(grid,),
        in_specs=[
            pl.BlockSpec((TB, d_in), lambda i: (i, 0)),
            _full((1, d_in)),
            _full((1, d_in)),
            _full((1, d_in)),
            _full((1, d_in)),
            _full((1, d_in)),
            _full((1, 1)),
        ],
        out_specs=pl.BlockSpec((TB, 1), lambda i: (i, 0)),
        out_shape=jax.ShapeDtypeStruct((B, 1), jnp.float32),
    )(h, s, q, g, bt, wo, bo)


def kernel(xb, emb_tables, W1, b1, g1, bt1, W2, b2, g2, bt2, W3, b3, g3, bt3, Wo, bo):
    cats = xb[:, N_NUM:N_NUM + N_CAT].astype(jnp.int32)
    gidx = (cats + (jnp.arange(N_CAT, dtype=jnp.int32) * VOCAB)[None, :]).reshape(-1)
    table = jnp.pad(emb_tables.reshape(N_CAT * VOCAB, EMB),
                    ((0, 0), (0, EMBP - EMB)))

    emb = _gather(table, gidx).reshape(B, N_CAT * EMBP)
    xnum = xb[:, :N_NUM]

    h1, s1, q1 = _layer1(xnum, emb, W1[:N_NUM], W1[N_NUM:], b1.reshape(1, -1))
    h2, s2, q2 = _mid(h1, s1, q1, g1.reshape(1, -1), bt1.reshape(1, -1), W2,
                      b2.reshape(1, -1))
    h3, s3, q3 = _mid(h2, s2, q2, g2.reshape(1, -1), bt2.reshape(1, -1), W3,
                      b3.reshape(1, -1))
    out = _final(h3, s3, q3, g3.reshape(1, -1), bt3.reshape(1, -1),
                 Wo.reshape(1, -1), bo.reshape(1, 1))
    return out.reshape(B)


# speedup vs baseline: 1.1470x; 1.0399x over previous
import functools

import jax
import jax.numpy as jnp
from jax import lax
from jax.experimental import pallas as pl
from jax.experimental.pallas import tpu as pltpu
from jax.experimental.pallas import tpu_sc as plsc

B = 4096
N_NUM = 13
N_CAT = 26
VOCAB = 1000
EMB = 50
R = B * N_CAT
EMBP = 64
TB = 1024
EPS = 1e-5


def _build_gather():
    info = plsc.get_sparse_core_info()
    nc, ns = info.num_cores, info.num_subcores
    nw = nc * ns
    per_w = R // nw
    ch = 128
    nch = per_w // ch
    assert per_w * nw == R and nch * ch == per_w

    mesh = plsc.VectorSubcoreMesh(core_axis_name="c", subcore_axis_name="s")

    @functools.partial(
        pl.kernel,
        out_type=jax.ShapeDtypeStruct((R, EMBP), jnp.float32),
        mesh=mesh,
        compiler_params=pltpu.CompilerParams(use_tc_tiling_on_sc=False),
        scratch_types=[
            pltpu.VMEM((per_w,), jnp.int32),
            pltpu.VMEM((ch, EMBP), jnp.float32),
            pltpu.VMEM((ch, EMBP), jnp.float32),
            pltpu.SemaphoreType.DMA,
            pltpu.SemaphoreType.DMA,
        ],
    )
    def gather(table_hbm, idx_hbm, out_hbm, idx_v, rows_a, rows_b, sem_a, sem_b):
        wid = lax.axis_index("s") * nc + lax.axis_index("c")
        base = wid * per_w
        pltpu.sync_copy(idx_hbm.at[pl.ds(base, per_w)], idx_v)

        def body(i, carry):
            c0 = i * 2
            d_a = pltpu.async_copy(
                table_hbm.at[idx_v.at[pl.ds(c0 * ch, ch)]], rows_a, sem_a)
            d_b = pltpu.async_copy(
                table_hbm.at[idx_v.at[pl.ds((c0 + 1) * ch, ch)]], rows_b, sem_b)
            d_a.wait()
            pltpu.sync_copy(rows_a, out_hbm.at[pl.ds(base + c0 * ch, ch)])
            d_b.wait()
            pltpu.sync_copy(rows_b, out_hbm.at[pl.ds(base + (c0 + 1) * ch, ch)])
            return carry

        lax.fori_loop(0, nch // 2, body, 0)

    return gather


_gather_cache = []


def _gather(table, gidx):
    if not _gather_cache:
        _gather_cache.append(_build_gather())
    return _gather_cache[0](table, gidx)


def _layer1_body(xnum_ref, emb_ref, w1a_ref, w1b_ref, b1_ref, h_ref, s_ref, q_ref,
                 wp_ref):
    i = pl.program_id(0)

    @pl.when(i == 0)
    def _():
        wp_ref[...] = jnp.zeros_like(wp_ref)
        for f in range(N_CAT):
            wp_ref[pl.ds(f * EMBP, EMB), :] = (
                w1b_ref[pl.ds(f * EMB, EMB), :].astype(jnp.bfloat16))

    h = (jnp.dot(xnum_ref[...].astype(jnp.bfloat16),
                 w1a_ref[...].astype(jnp.bfloat16),
                 preferred_element_type=jnp.float32)
         + jnp.dot(emb_ref[...].astype(jnp.bfloat16), wp_ref[...],
                   preferred_element_type=jnp.float32)
         + b1_ref[...])
    h_ref[...] = h

    @pl.when(i == 0)
    def _():
        s_ref[...] = jnp.zeros_like(s_ref)
        q_ref[...] = jnp.zeros_like(q_ref)

    s_ref[...] += jnp.sum(h, axis=0, keepdims=True)
    q_ref[...] += jnp.sum(h * h, axis=0, keepdims=True)


def _mid_body(h_ref, s_in, q_in, g_ref, bt_ref, w_ref, b_ref, o_ref, s_ref, q_ref):
    i = pl.program_id(0)
    mu = s_in[...] * (1.0 / B)
    var = q_in[...] * (1.0 / B) - mu * mu
    a = g_ref[...] * lax.rsqrt(var + EPS)
    c = bt_ref[...] - mu * a
    x = jnp.maximum(h_ref[...] * a + c, 0.0)
    h = jnp.dot(x.astype(jnp.bfloat16), w_ref[...].astype(jnp.bfloat16),
                preferred_element_type=jnp.float32) + b_ref[...]
    o_ref[...] = h

    @pl.when(i == 0)
    def _():
        s_ref[...] = jnp.zeros_like(s_ref)
        q_ref[...] = jnp.zeros_like(q_ref)

    s_ref[...] += jnp.sum(h, axis=0, keepdims=True)
    q_ref[...] += jnp.sum(h * h, axis=0, keepdims=True)


def _final_body(h_ref, s_in, q_in, g_ref, bt_ref, wo_ref, bo_ref, o_ref):
    mu = s_in[...] * (1.0 / B)
    var = q_in[...] * (1.0 / B) - mu * mu
    a = g_ref[...] * lax.rsqrt(var + EPS)
    c = bt_ref[...] - mu * a
    x = jnp.maximum(h_ref[...] * a + c, 0.0)
    o_ref[...] = jnp.sum(x * wo_ref[...], axis=1, keepdims=True) + bo_ref[0, 0]


def _full(shape):
    return pl.BlockSpec(shape, lambda i: (0, 0))


def _layer1(xnum, emb, w1a, w1b, b1):
    d = w1a.shape[1]
    grid = B // TB
    return pl.pallas_call(
        _layer1_body,
        grid=(grid,),
        in_specs=[
            pl.BlockSpec((TB, N_NUM), lambda i: (i, 0)),
            pl.BlockSpec((TB, N_CAT * EMBP), lambda i: (i, 0)),
            _full(w1a.shape),
            _full(w1b.shape),
            _full((1, d)),
        ],
        out_specs=[
            pl.BlockSpec((TB, d), lambda i: (i, 0)),
            _full((1, d)),
            _full((1, d)),
        ],
        out_shape=[
            jax.ShapeDtypeStruct((B, d), jnp.float32),
            jax.ShapeDtypeStruct((1, d), jnp.float32),
            jax.ShapeDtypeStruct((1, d), jnp.float32),
        ],
        scratch_shapes=[pltpu.VMEM((N_CAT * EMBP, d), jnp.bfloat16)],
    )(xnum, emb, w1a, w1b, b1)


def _mid(h, s, q, g, bt, w, b):
    d_in, d_out = w.shape
    grid = B // TB
    return pl.pallas_call(
        _mid_body,
        grid=(grid,),
        in_specs=[
            pl.BlockSpec((TB, d_in), lambda i: (i, 0)),
            _full((1, d_in)),
            _full((1, d_in)),
            _full((1, d_in)),
            _full((1, d_in)),
            _full((d_in, d_out)),
            _full((1, d_out)),
        ],
        out_specs=[
            pl.BlockSpec((TB, d_out), lambda i: (i, 0)),
            _full((1, d_out)),
            _full((1, d_out)),
        ],
        out_shape=[
            jax.ShapeDtypeStruct((B, d_out), jnp.float32),
            jax.ShapeDtypeStruct((1, d_out), jnp.float32),
            jax.ShapeDtypeStruct((1, d_out), jnp.float32),
        ],
    )(h, s, q, g, bt, w, b)


def _final(h, s, q, g, bt, wo, bo):
    d_in = h.shape[1]
    grid = B // TB
    return pl.pallas_call(
        _final_body,
        grid=(grid,),
        in_specs=[
            pl.BlockSpec((TB, d_in), lambda i: (i, 0)),
            _full((1, d_in)),
            _full((1, d_in)),
            _full((1, d_in)),
            _full((1, d_in)),
            _full((1, d_in)),
            _full((1, 1)),
        ],
        out_specs=pl.BlockSpec((TB, 1), lambda i: (i, 0)),
        out_shape=jax.ShapeDtypeStruct((B, 1), jnp.float32),
    )(h, s, q, g, bt, wo, bo)


def kernel(xb, emb_tables, W1, b1, g1, bt1, W2, b2, g2, bt2, W3, b3, g3, bt3, Wo, bo):
    cats = xb[:, N_NUM:N_NUM + N_CAT].astype(jnp.int32)
    gidx = (cats + (jnp.arange(N_CAT, dtype=jnp.int32) * VOCAB)[None, :]).reshape(-1)
    table = jnp.pad(emb_tables.reshape(N_CAT * VOCAB, EMB),
                    ((0, 0), (0, EMBP - EMB)))

    emb = _gather(table, gidx).reshape(B, N_CAT * EMBP)
    xnum = xb[:, :N_NUM]

    h1, s1, q1 = _layer1(xnum, emb, W1[:N_NUM], W1[N_NUM:], b1.reshape(1, -1))
    h2, s2, q2 = _mid(h1, s1, q1, g1.reshape(1, -1), bt1.reshape(1, -1), W2,
                      b2.reshape(1, -1))
    h3, s3, q3 = _mid(h2, s2, q2, g2.reshape(1, -1), bt2.reshape(1, -1), W3,
                      b3.reshape(1, -1))
    out = _final(h3, s3, q3, g3.reshape(1, -1), bt3.reshape(1, -1),
                 Wo.reshape(1, -1), bo.reshape(1, 1))
    return out.reshape(B)


# speedup vs baseline: 1.1687x; 1.0190x over previous
import functools

import jax
import jax.numpy as jnp
from jax import lax
from jax.experimental import pallas as pl
from jax.experimental.pallas import tpu as pltpu
from jax.experimental.pallas import tpu_sc as plsc

B = 4096
N_NUM = 13
N_CAT = 26
VOCAB = 1000
EMB = 50
R = B * N_CAT
EMBP = 64
TB = 1024
EPS = 1e-5


def _build_gather():
    info = plsc.get_sparse_core_info()
    nc, ns = info.num_cores, info.num_subcores
    nw = nc * ns
    per_w = R // nw
    ch = 128
    nch = per_w // ch
    assert per_w * nw == R and nch * ch == per_w

    mesh = plsc.VectorSubcoreMesh(core_axis_name="c", subcore_axis_name="s")

    @functools.partial(
        pl.kernel,
        out_type=jax.ShapeDtypeStruct((R, EMBP), jnp.float32),
        mesh=mesh,
        compiler_params=pltpu.CompilerParams(use_tc_tiling_on_sc=False),
        scratch_types=[
            pltpu.VMEM((per_w,), jnp.int32),
            pltpu.VMEM((ch, EMBP), jnp.float32),
            pltpu.VMEM((ch, EMBP), jnp.float32),
            pltpu.VMEM((ch, EMBP), jnp.float32),
            pltpu.VMEM((ch, EMBP), jnp.float32),
            pltpu.SemaphoreType.DMA,
            pltpu.SemaphoreType.DMA,
            pltpu.SemaphoreType.DMA,
            pltpu.SemaphoreType.DMA,
        ],
    )
    def gather(table_hbm, idx_hbm, out_hbm, idx_v, rows_a, rows_b, rows_c,
               rows_d, sem_a, sem_b, sem_c, sem_d):
        wid = lax.axis_index("s") * nc + lax.axis_index("c")
        base = wid * per_w
        pltpu.sync_copy(idx_hbm.at[pl.ds(base, per_w)], idx_v)
        rows = (rows_a, rows_b, rows_c, rows_d)
        sems = (sem_a, sem_b, sem_c, sem_d)

        def quad(i, carry):
            c0 = i * 4
            ds = [pltpu.async_copy(
                table_hbm.at[idx_v.at[pl.ds((c0 + k) * ch, ch)]],
                rows[k], sems[k]) for k in range(4)]
            for k in range(4):
                ds[k].wait()
                pltpu.sync_copy(rows[k],
                                out_hbm.at[pl.ds(base + (c0 + k) * ch, ch)])
            return carry

        lax.fori_loop(0, nch // 4, quad, 0)
        c0 = (nch // 4) * 4
        ds = [pltpu.async_copy(
            table_hbm.at[idx_v.at[pl.ds((c0 + k) * ch, ch)]],
            rows[k], sems[k]) for k in range(nch % 4)]
        for k in range(nch % 4):
            ds[k].wait()
            pltpu.sync_copy(rows[k],
                            out_hbm.at[pl.ds(base + (c0 + k) * ch, ch)])

    return gather


_gather_cache = []


def _gather(table, gidx):
    if not _gather_cache:
        _gather_cache.append(_build_gather())
    return _gather_cache[0](table, gidx)


def _layer1_body(xnum_ref, emb_ref, w1a_ref, w1b_ref, b1_ref, h_ref, s_ref, q_ref,
                 wp_ref):
    i = pl.program_id(0)

    @pl.when(i == 0)
    def _():
        wp_ref[...] = jnp.zeros_like(wp_ref)
        for f in range(N_CAT):
            wp_ref[pl.ds(f * EMBP, EMB), :] = (
                w1b_ref[pl.ds(f * EMB, EMB), :].astype(jnp.bfloat16))

    h = (jnp.dot(xnum_ref[...].astype(jnp.bfloat16),
                 w1a_ref[...].astype(jnp.bfloat16),
                 preferred_element_type=jnp.float32)
         + jnp.dot(emb_ref[...].astype(jnp.bfloat16), wp_ref[...],
                   preferred_element_type=jnp.float32)
         + b1_ref[...])
    h_ref[...] = h

    @pl.when(i == 0)
    def _():
        s_ref[...] = jnp.zeros_like(s_ref)
        q_ref[...] = jnp.zeros_like(q_ref)

    s_ref[...] += jnp.sum(h, axis=0, keepdims=True)
    q_ref[...] += jnp.sum(h * h, axis=0, keepdims=True)


def _mid_body(h_ref, s_in, q_in, g_ref, bt_ref, w_ref, b_ref, o_ref, s_ref, q_ref):
    i = pl.program_id(0)
    mu = s_in[...] * (1.0 / B)
    var = q_in[...] * (1.0 / B) - mu * mu
    a = g_ref[...] * lax.rsqrt(var + EPS)
    c = bt_ref[...] - mu * a
    x = jnp.maximum(h_ref[...] * a + c, 0.0)
    h = jnp.dot(x.astype(jnp.bfloat16), w_ref[...].astype(jnp.bfloat16),
                preferred_element_type=jnp.float32) + b_ref[...]
    o_ref[...] = h

    @pl.when(i == 0)
    def _():
        s_ref[...] = jnp.zeros_like(s_ref)
        q_ref[...] = jnp.zeros_like(q_ref)

    s_ref[...] += jnp.sum(h, axis=0, keepdims=True)
    q_ref[...] += jnp.sum(h * h, axis=0, keepdims=True)


def _final_body(h_ref, s_in, q_in, g_ref, bt_ref, wo_ref, bo_ref, o_ref):
    mu = s_in[...] * (1.0 / B)
    var = q_in[...] * (1.0 / B) - mu * mu
    a = g_ref[...] * lax.rsqrt(var + EPS)
    c = bt_ref[...] - mu * a
    x = jnp.maximum(h_ref[...] * a + c, 0.0)
    o_ref[...] = jnp.sum(x * wo_ref[...], axis=1, keepdims=True) + bo_ref[0, 0]


def _full(shape):
    return pl.BlockSpec(shape, lambda i: (0, 0))


def _layer1(xnum, emb, w1a, w1b, b1):
    d = w1a.shape[1]
    grid = B // TB
    return pl.pallas_call(
        _layer1_body,
        grid=(grid,),
        in_specs=[
            pl.BlockSpec((TB, N_NUM), lambda i: (i, 0)),
            pl.BlockSpec((TB, N_CAT * EMBP), lambda i: (i, 0)),
            _full(w1a.shape),
            _full(w1b.shape),
            _full((1, d)),
        ],
        out_specs=[
            pl.BlockSpec((TB, d), lambda i: (i, 0)),
            _full((1, d)),
            _full((1, d)),
        ],
        out_shape=[
            jax.ShapeDtypeStruct((B, d), jnp.float32),
            jax.ShapeDtypeStruct((1, d), jnp.float32),
            jax.ShapeDtypeStruct((1, d), jnp.float32),
        ],
        scratch_shapes=[pltpu.VMEM((N_CAT * EMBP, d), jnp.bfloat16)],
    )(xnum, emb, w1a, w1b, b1)


def _mid(h, s, q, g, bt, w, b):
    d_in, d_out = w.shape
    grid = B // TB
    return pl.pallas_call(
        _mid_body,
        grid=(grid,),
        in_specs=[
            pl.BlockSpec((TB, d_in), lambda i: (i, 0)),
            _full((1, d_in)),
            _full((1, d_in)),
            _full((1, d_in)),
            _full((1, d_in)),
            _full((d_in, d_out)),
            _full((1, d_out)),
        ],
        out_specs=[
            pl.BlockSpec((TB, d_out), lambda i: (i, 0)),
            _full((1, d_out)),
            _full((1, d_out)),
        ],
        out_shape=[
            jax.ShapeDtypeStruct((B, d_out), jnp.float32),
            jax.ShapeDtypeStruct((1, d_out), jnp.float32),
            jax.ShapeDtypeStruct((1, d_out), jnp.float32),
        ],
    )(h, s, q, g, bt, w, b)


def _final(h, s, q, g, bt, wo, bo):
    d_in = h.shape[1]
    grid = B // TB
    return pl.pallas_call(
        _final_body,
        grid=(grid,),
        in_specs=[
            pl.BlockSpec((TB, d_in), lambda i: (i, 0)),
            _full((1, d_in)),
            _full((1, d_in)),
            _full((1, d_in)),
            _full((1, d_in)),
            _full((1, d_in)),
            _full((1, 1)),
        ],
        out_specs=pl.BlockSpec((TB, 1), lambda i: (i, 0)),
        out_shape=jax.ShapeDtypeStruct((B, 1), jnp.float32),
    )(h, s, q, g, bt, wo, bo)


def kernel(xb, emb_tables, W1, b1, g1, bt1, W2, b2, g2, bt2, W3, b3, g3, bt3, Wo, bo):
    cats = xb[:, N_NUM:N_NUM + N_CAT].astype(jnp.int32)
    gidx = (cats + (jnp.arange(N_CAT, dtype=jnp.int32) * VOCAB)[None, :]).reshape(-1)
    table = jnp.pad(emb_tables.reshape(N_CAT * VOCAB, EMB),
                    ((0, 0), (0, EMBP - EMB)))

    emb = _gather(table, gidx).reshape(B, N_CAT * EMBP)
    xnum = xb[:, :N_NUM]

    h1, s1, q1 = _layer1(xnum, emb, W1[:N_NUM], W1[N_NUM:], b1.reshape(1, -1))
    h2, s2, q2 = _mid(h1, s1, q1, g1.reshape(1, -1), bt1.reshape(1, -1), W2,
                      b2.reshape(1, -1))
    h3, s3, q3 = _mid(h2, s2, q2, g2.reshape(1, -1), bt2.reshape(1, -1), W3,
                      b3.reshape(1, -1))
    out = _final(h3, s3, q3, g3.reshape(1, -1), bt3.reshape(1, -1),
                 Wo.reshape(1, -1), bo.reshape(1, 1))
    return out.reshape(B)
